# Initial kernel scaffold; baseline (speedup 1.0000x reference)
import jax
import jax.numpy as jnp
from jax.experimental import pallas as pl


def kernel(X, Z, adj_e, adj_v, T, batch, W1, p1, b1, W2, p2, b2, W3, p3, b3, Wl, bl):
    raise NotImplementedError("write your pallas kernel here")


# speedup vs baseline: 2.1033x; 2.1033x over previous
import jax
import jax.numpy as jnp
from jax.experimental import pallas as pl
from jax.experimental.pallas import tpu as pltpu

_N, _E, _G = 1024, 2048, 32
_NC = 32
_EBLK = 256


def _matmul(a, b):
    return jax.lax.dot_general(
        a, b, (((1,), (0,)), ((), ())), preferred_element_type=jnp.float32
    )


def _gcn_body(X_ref, Z_ref, adj_e_ref, adj_v_ref, T_ref, Tt_ref, Pt_ref,
              W1_ref, p1_ref, b1_ref, W2_ref, p2_ref, b2_ref,
              W3_ref, p3_ref, b3_ref, Wl_ref, bl_ref,
              out_ref, zh2_s):
    X = X_ref[...]
    Z = Z_ref[...]
    T = T_ref[...]
    Tt = Tt_ref[...]

    rows = jax.lax.broadcasted_iota(jnp.int32, (_N, _N), 0)
    cols = jax.lax.broadcasted_iota(jnp.int32, (_N, _N), 1)
    diag_v = rows == cols

    d1 = jnp.sum(Z * p1_ref[...], axis=1, keepdims=True)
    Hw1 = _matmul(X, W1_ref[...])
    mult1 = _matmul(T, d1 * Tt)
    A1 = jnp.where(diag_v, adj_v_ref[...], mult1 * adj_v_ref[...])
    Xh1 = jnp.maximum(_matmul(A1, Hw1) + b1_ref[...], 0.0)

    Zr = jnp.maximum(Z, 0.0)
    HeW = _matmul(Zr, W2_ref[...])
    d2 = jnp.sum(Xh1 * p2_ref[...], axis=1, keepdims=True)
    Tse = d2 * T
    for k in range(_E // _EBLK):
        r0 = k * _EBLK
        mult_blk = _matmul(Tt_ref[pl.ds(r0, _EBLK), :], Tse)
        adj_blk = adj_e_ref[pl.ds(r0, _EBLK), :]
        rr = jax.lax.broadcasted_iota(jnp.int32, (_EBLK, _E), 0)
        cc = jax.lax.broadcasted_iota(jnp.int32, (_EBLK, _E), 1)
        A_blk = jnp.where(cc == rr + r0, adj_blk, mult_blk * adj_blk)
        zh2_s[pl.ds(r0, _EBLK), :] = jnp.maximum(
            _matmul(A_blk, HeW) + b2_ref[...], 0.0)
    Zh2 = zh2_s[...]

    d3 = jnp.sum(Zh2 * p3_ref[...], axis=1, keepdims=True)
    Hw3 = _matmul(Xh1, W3_ref[...])
    mult3 = _matmul(T, d3 * Tt)
    A3 = jnp.where(diag_v, adj_v_ref[...], mult3 * adj_v_ref[...])
    Xh3 = jnp.maximum(_matmul(A3, Hw3) + b3_ref[...], 0.0)

    Pt = Pt_ref[...]
    pooled = _matmul(Pt, Xh3)
    counts = jnp.sum(Pt, axis=1, keepdims=True)
    mean = pooled / jnp.maximum(counts, 1.0)
    out_ref[...] = _matmul(mean, Wl_ref[...]) + bl_ref[...]


def kernel(X, Z, adj_e, adj_v, T, batch, W1, p1, b1, W2, p2, b2, W3, p3, b3, Wl, bl):
    Tt = T.T
    Pt = (batch.astype(jnp.int32)[None, :]
          == jnp.arange(_G, dtype=jnp.int32)[:, None]).astype(jnp.float32)
    return pl.pallas_call(
        _gcn_body,
        out_shape=jax.ShapeDtypeStruct((_G, _NC), jnp.float32),
        scratch_shapes=[pltpu.VMEM((_E, 16), jnp.float32)],
    )(X, Z, adj_e, adj_v, T, Tt, Pt,
      W1, p1, b1.reshape(1, -1), W2, p2, b2.reshape(1, -1),
      W3, p3, b3.reshape(1, -1), Wl, bl.reshape(1, -1))
